# baseline (speedup 1.0000x reference)
import functools

import jax
import jax.numpy as jnp
from jax.experimental import pallas as pl
from jax.experimental.pallas import tpu as pltpu

EMB = 64
DIN = 2 * EMB
HID = 4 * EMB

O_N = 10000
T_E = 160000

BE = 1000
BN = 1000


def _edge_body(g_ref, pred_ref, w1p_ref, b1_ref, w2_ref, b2_ref, out_ref):
    h1 = jnp.maximum(g_ref[...] + pred_ref[...] @ w1p_ref[...] + b1_ref[...], 0.0)
    out_ref[...] = jnp.maximum(h1 @ w2_ref[...] + b2_ref[...], 0.0)


def _edge_mlp(g, pred, w1p, b1, w2, b2):
    t = g.shape[0]
    dout = w2.shape[1]
    din_p = pred.shape[1]
    return pl.pallas_call(
        _edge_body,
        grid=(t // BE,),
        in_specs=[
            pl.BlockSpec((BE, HID), lambda i: (i, 0)),
            pl.BlockSpec((BE, din_p), lambda i: (i, 0)),
            pl.BlockSpec((din_p, HID), lambda i: (0, 0)),
            pl.BlockSpec((1, HID), lambda i: (0, 0)),
            pl.BlockSpec((HID, dout), lambda i: (0, 0)),
            pl.BlockSpec((1, dout), lambda i: (0, 0)),
        ],
        out_specs=pl.BlockSpec((BE, dout), lambda i: (i, 0)),
        out_shape=jax.ShapeDtypeStruct((t, dout), jnp.float32),
    )(g, pred, w1p, b1, w2, b2)


def _dense_body(x_ref, w_ref, out_ref):
    out_ref[...] = x_ref[...] @ w_ref[...]


def _dense1(x, w, block=BN):
    n, din = x.shape
    dout = w.shape[1]
    return pl.pallas_call(
        _dense_body,
        grid=(n // block,),
        in_specs=[
            pl.BlockSpec((block, din), lambda i: (i, 0)),
            pl.BlockSpec((din, dout), lambda i: (0, 0)),
        ],
        out_specs=pl.BlockSpec((block, dout), lambda i: (i, 0)),
        out_shape=jax.ShapeDtypeStruct((n, dout), jnp.float32),
    )(x, w)


def _mlp2_body(x_ref, w1_ref, b1_ref, w2_ref, b2_ref, out_ref, *, relu1, relu2):
    h = x_ref[...] @ w1_ref[...] + b1_ref[...]
    if relu1:
        h = jnp.maximum(h, 0.0)
    o = h @ w2_ref[...] + b2_ref[...]
    if relu2:
        o = jnp.maximum(o, 0.0)
    out_ref[...] = o


def _mlp2(x, w1, b1, w2, b2, relu1=True, relu2=True, block=BN):
    n, din = x.shape
    dh = w1.shape[1]
    dout = w2.shape[1]
    return pl.pallas_call(
        functools.partial(_mlp2_body, relu1=relu1, relu2=relu2),
        grid=(n // block,),
        in_specs=[
            pl.BlockSpec((block, din), lambda i: (i, 0)),
            pl.BlockSpec((din, dh), lambda i: (0, 0)),
            pl.BlockSpec((1, dh), lambda i: (0, 0)),
            pl.BlockSpec((dh, dout), lambda i: (0, 0)),
            pl.BlockSpec((1, dout), lambda i: (0, 0)),
        ],
        out_specs=pl.BlockSpec((block, dout), lambda i: (i, 0)),
        out_shape=jax.ShapeDtypeStruct((n, dout), jnp.float32),
    )(x, w1, b1, w2, b2)


def _head_body(x_ref, h1_ref, c1_ref, h2_ref, c2_ref, wm_ref, bm_ref,
               wv_ref, bv_ref, mu_ref, lv_ref):
    h = jnp.maximum(x_ref[...] @ h1_ref[...] + c1_ref[...], 0.0)
    be = jnp.maximum(h @ h2_ref[...] + c2_ref[...], 0.0)
    mu_ref[...] = be @ wm_ref[...] + bm_ref[...]
    lv_ref[...] = be @ wv_ref[...] + bv_ref[...]


def _head(x, mv_params, mean_p, var_p):
    (h1, c1), (h2, c2) = mv_params
    wm, bm = mean_p
    wv, bv = var_p
    c1 = c1.reshape(1, -1)
    c2 = c2.reshape(1, -1)
    bm = bm.reshape(1, -1)
    bv = bv.reshape(1, -1)
    n = x.shape[0]
    return pl.pallas_call(
        _head_body,
        grid=(n // BN,),
        in_specs=[
            pl.BlockSpec((BN, 2 * EMB), lambda i: (i, 0)),
            pl.BlockSpec((2 * EMB, HID), lambda i: (0, 0)),
            pl.BlockSpec((1, HID), lambda i: (0, 0)),
            pl.BlockSpec((HID, 2 * EMB), lambda i: (0, 0)),
            pl.BlockSpec((1, 2 * EMB), lambda i: (0, 0)),
            pl.BlockSpec((2 * EMB, EMB), lambda i: (0, 0)),
            pl.BlockSpec((1, EMB), lambda i: (0, 0)),
            pl.BlockSpec((2 * EMB, EMB), lambda i: (0, 0)),
            pl.BlockSpec((1, EMB), lambda i: (0, 0)),
        ],
        out_specs=[
            pl.BlockSpec((BN, EMB), lambda i: (i, 0)),
            pl.BlockSpec((BN, EMB), lambda i: (i, 0)),
        ],
        out_shape=[
            jax.ShapeDtypeStruct((n, EMB), jnp.float32),
            jax.ShapeDtypeStruct((n, EMB), jnp.float32),
        ],
    )(x, h1, c1, h2, c2, wm, bm, wv, bv)


def _gconv(layer, obj_vecs, pred_vecs, s, o, inv_counts, dout_p):
    net1, net2 = layer
    (w1, b1), (w2, b2) = net1
    (v1, c1), (v2, c2) = net2
    din = obj_vecs.shape[1]
    dout_all = w2.shape[1]

    w1s = w1[:din]
    w1p = w1[din:2 * din]
    w1o = w1[2 * din:]

    ab = _dense1(obj_vecs, jnp.concatenate([w1s, w1o], axis=1))
    a = ab[:, :HID]
    b = ab[:, HID:]

    g = jnp.take(a, s, axis=0) + jnp.take(b, o, axis=0)
    new = _edge_mlp(g, pred_vecs, w1p, b1.reshape(1, -1), w2, b2.reshape(1, -1))

    new_s = new[:, :HID]
    new_p = new[:, HID:HID + dout_p]
    new_o = new[:, HID + dout_p:]

    pooled = jnp.zeros((O_N, HID), jnp.float32).at[s].add(new_s).at[o].add(new_o)
    pooled = pooled * inv_counts

    new_obj = _mlp2(pooled, v1, c1.reshape(1, -1), v2, c2.reshape(1, -1))
    return new_obj, new_p


def kernel(objs, triples, boxes_gt, shapes_gt, attributes, params):
    s = triples[:, 0]
    p = triples[:, 1]
    o = triples[:, 2]

    counts = jnp.zeros((O_N,), jnp.float32).at[s].add(1.0).at[o].add(1.0)
    inv_counts = (1.0 / jnp.maximum(counts, 1.0))[:, None]

    obj_box = jnp.take(params['emb_obj_box'], objs, axis=0)
    obj_shape = jnp.take(params['emb_obj_shape'], objs, axis=0)
    pred_box = jnp.take(params['emb_pred_box'], p, axis=0)
    pred_shape = jnp.take(params['emb_pred_shape'], p, axis=0)
    wbe, bbe = params['box_embeddings']
    wse, bse = params['shape_embeddings']
    box_vecs = boxes_gt @ wbe + bbe
    shape_vecs = shapes_gt @ wse + bse

    ob = jnp.concatenate([obj_box, box_vecs], axis=1)
    osh = jnp.concatenate([obj_shape, shape_vecs], axis=1)
    pb = pred_box
    ps = pred_shape

    for layer in params['gconv_ec_box']:
        ob, pb = _gconv(layer, ob, pb, s, o, inv_counts, DIN)
    for layer in params['gconv_ec_shape']:
        osh, ps = _gconv(layer, osh, ps, s, o, inv_counts, DIN)

    obj_sh = jnp.concatenate([ob, osh], axis=1)
    pred_sh = jnp.concatenate([pb, ps], axis=1)
    for layer in params['gconv_shared']:
        obj_sh, pred_sh = _gconv(layer, obj_sh, pred_sh, s, o, inv_counts, 2 * DIN)

    ob2 = obj_sh[:, :2 * EMB]
    osh2 = obj_sh[:, 2 * EMB:]

    mu_box, logvar_box = _head(ob2, params['box_mean_var'],
                               params['box_mean'][0], params['box_var'][0])
    mu_shape, logvar_shape = _head(osh2, params['shape_mean_var'],
                                   params['shape_mean'][0], params['shape_var'][0])
    return mu_box, logvar_box, mu_shape, logvar_shape


# speedup vs baseline: 2.2013x; 2.2013x over previous
import functools

import jax
import jax.numpy as jnp
from jax import lax
from jax.experimental import pallas as pl
from jax.experimental.pallas import tpu as pltpu
from jax.experimental.pallas import tpu_sc as plsc

EMB = 64
DIN = 2 * EMB
HID = 4 * EMB

O_N = 10000
T_E = 160000
O_PAD = 10240
NROWS_T = O_PAD // 16

BE = 1000
BN = 1000

C = 128
NCH_G = T_E // C
NCH_S = 2 * T_E // C
NCH_C = T_E // C

@functools.cache
def _mesh():
    return plsc.VectorSubcoreMesh(core_axis_name="c", subcore_axis_name="s",
                                  num_cores=2, num_subcores=16)


def _sc_gather2(tab, ia, ib):
    @functools.partial(
        pl.kernel,
        out_type=(jax.ShapeDtypeStruct((T_E, HID), jnp.float32),
                  jax.ShapeDtypeStruct((T_E, HID), jnp.float32)),
        mesh=_mesh(),
        scratch_types=[
            pltpu.VMEM((C,), jnp.int32),
            pltpu.VMEM((C,), jnp.int32),
            pltpu.VMEM((C, HID), jnp.float32),
            pltpu.VMEM((C, HID), jnp.float32),
            pltpu.SemaphoreType.DMA,
            pltpu.SemaphoreType.DMA,
        ],
    )
    def k(tab_h, ia_h, ib_h, g1_h, g2_h, ia_v, ib_v, buf1, buf2, sem1, sem2):
        wid = lax.axis_index("s") * 2 + lax.axis_index("c")

        def body(kk, carry):
            c = wid + kk * 32

            @pl.when(c < NCH_G)
            def _():
                base = c * C
                pltpu.sync_copy(ia_h.at[pl.ds(base, C)], ia_v)
                pltpu.sync_copy(ib_h.at[pl.ds(base, C)], ib_v)
                d1 = pltpu.async_copy(tab_h.at[ia_v], buf1, sem1)
                d2 = pltpu.async_copy(tab_h.at[ib_v], buf2, sem2)
                d1.wait()
                d2.wait()
                pltpu.sync_copy(buf1, g1_h.at[pl.ds(base, C)])
                pltpu.sync_copy(buf2, g2_h.at[pl.ds(base, C)])

            return carry

        lax.fori_loop(0, (NCH_G + 31) // 32, body, 0)

    return k(tab, ia, ib)


def _sc_scatter(scat4, node_il, src_both, zeros_init):
    @functools.partial(
        pl.kernel,
        out_type=jax.ShapeDtypeStruct((2, O_PAD, 128), jnp.float32),
        mesh=_mesh(),
        scratch_types=[
            pltpu.VMEM((C,), jnp.int32),
            pltpu.VMEM((C,), jnp.int32),
            pltpu.VMEM((C, 128), jnp.float32),
            pltpu.VMEM_SHARED((O_PAD, 128), jnp.float32),
            pltpu.SemaphoreType.DMA,
        ],
    )
    def k(scat_h, node_h, src_h, z_h, out_h, ni_v, si_v, buf, acc, sem):
        cid = lax.axis_index("c")
        sid = lax.axis_index("s")
        pltpu.sync_copy(z_h, acc.at[pl.ds(sid * NROWS_T, NROWS_T)])
        plsc.subcore_barrier()

        def body(kk, carry):
            c = sid + kk * 16

            @pl.when(c < NCH_S)
            def _():
                base = c * C
                pltpu.sync_copy(node_h.at[pl.ds(base, C)], ni_v)
                pltpu.sync_copy(src_h.at[cid, pl.ds(base, C)], si_v)
                pltpu.async_copy(scat_h.at[si_v], buf, sem).wait()
                pltpu.sync_copy(buf, acc.at[ni_v], add=True)

            return carry

        lax.fori_loop(0, (NCH_S + 15) // 16, body, 0)
        plsc.subcore_barrier()
        pltpu.sync_copy(acc.at[pl.ds(sid * NROWS_T, NROWS_T)],
                        out_h.at[cid, pl.ds(sid * NROWS_T, NROWS_T)])

    return k(scat4, node_il, src_both, zeros_init)


def _sc_counts(node_il, ones_blk, zeros_init):
    @functools.partial(
        pl.kernel,
        out_type=jax.ShapeDtypeStruct((2, O_PAD, 128), jnp.float32),
        mesh=_mesh(),
        scratch_types=[
            pltpu.VMEM((C,), jnp.int32),
            pltpu.VMEM((C, 128), jnp.float32),
            pltpu.VMEM_SHARED((O_PAD, 128), jnp.float32),
        ],
    )
    def k(node_h, ones_h, z_h, out_h, ni_v, ones_v, acc):
        cid = lax.axis_index("c")
        sid = lax.axis_index("s")
        pltpu.sync_copy(ones_h, ones_v)
        pltpu.sync_copy(z_h, acc.at[pl.ds(sid * NROWS_T, NROWS_T)])
        plsc.subcore_barrier()

        def body(kk, carry):
            c = sid + kk * 16

            @pl.when(c < NCH_C)
            def _():
                base = cid * T_E + c * C
                pltpu.sync_copy(node_h.at[pl.ds(base, C)], ni_v)
                pltpu.sync_copy(ones_v, acc.at[ni_v], add=True)

            return carry

        lax.fori_loop(0, (NCH_C + 15) // 16, body, 0)
        plsc.subcore_barrier()
        pltpu.sync_copy(acc.at[pl.ds(sid * NROWS_T, NROWS_T)],
                        out_h.at[cid, pl.ds(sid * NROWS_T, NROWS_T)])

    return k(node_il, ones_blk, zeros_init)


def _ab2_body(x1_ref, w1_ref, x2_ref, w2_ref, c_ref, out_ref):
    out_ref[...] = (x1_ref[...] @ w1_ref[...] + x2_ref[...] @ w2_ref[...]
                    + c_ref[...])


def _ab2(x1, w1, x2, w2, crow):
    n = x1.shape[0]
    d1 = x1.shape[1]
    d2 = x2.shape[1]
    dout = w1.shape[1]
    return pl.pallas_call(
        _ab2_body,
        grid=(n // BN,),
        in_specs=[
            pl.BlockSpec((BN, d1), lambda i: (i, 0)),
            pl.BlockSpec((d1, dout), lambda i: (0, 0)),
            pl.BlockSpec((BN, d2), lambda i: (i, 0)),
            pl.BlockSpec((d2, dout), lambda i: (0, 0)),
            pl.BlockSpec((1, dout), lambda i: (0, 0)),
        ],
        out_specs=pl.BlockSpec((BN, dout), lambda i: (i, 0)),
        out_shape=jax.ShapeDtypeStruct((n, dout), jnp.float32),
    )(x1, w1, x2, w2, crow)


def _dense_body(x_ref, w_ref, out_ref):
    out_ref[...] = x_ref[...] @ w_ref[...]


def _dense1(x, w, block=BN):
    n, din = x.shape
    dout = w.shape[1]
    return pl.pallas_call(
        _dense_body,
        grid=(n // block,),
        in_specs=[
            pl.BlockSpec((block, din), lambda i: (i, 0)),
            pl.BlockSpec((din, dout), lambda i: (0, 0)),
        ],
        out_specs=pl.BlockSpec((block, dout), lambda i: (i, 0)),
        out_shape=jax.ShapeDtypeStruct((n, dout), jnp.float32),
    )(x, w)


def _edge_body(*refs, n_pred):
    g1_ref, g2_ref = refs[0], refs[1]
    pred_refs = refs[2:2 + 2 * n_pred]
    b1_ref, w2_ref, b2_ref, scat_ref, newp_ref = refs[2 + 2 * n_pred:]
    h1 = g1_ref[...] + g2_ref[...] + b1_ref[...]
    for i in range(n_pred):
        h1 = h1 + pred_refs[2 * i][...] @ pred_refs[2 * i + 1][...]
    h1 = jnp.maximum(h1, 0.0)
    h2 = jnp.maximum(h1 @ w2_ref[...] + b2_ref[...], 0.0)
    scat_ref[...] = h2[:, :2 * HID]
    newp_ref[...] = h2[:, 2 * HID:]


def _edge_mlp(g1, g2, preds, b1, w2r, b2r):
    dout = w2r.shape[1]
    dp = dout - 2 * HID
    n_pred = len(preds)
    in_specs = [
        pl.BlockSpec((BE, HID), lambda i: (i, 0)),
        pl.BlockSpec((BE, HID), lambda i: (i, 0)),
    ]
    args = [g1, g2]
    for pred, w1p in preds:
        dpi = pred.shape[1]
        in_specs.append(pl.BlockSpec((BE, dpi), lambda i: (i, 0)))
        in_specs.append(pl.BlockSpec((dpi, HID), lambda i: (0, 0)))
        args += [pred, w1p]
    in_specs += [
        pl.BlockSpec((1, HID), lambda i: (0, 0)),
        pl.BlockSpec((HID, dout), lambda i: (0, 0)),
        pl.BlockSpec((1, dout), lambda i: (0, 0)),
    ]
    args += [b1, w2r, b2r]
    return pl.pallas_call(
        functools.partial(_edge_body, n_pred=n_pred),
        grid=(T_E // BE,),
        in_specs=in_specs,
        out_specs=[
            pl.BlockSpec((BE, 2 * HID), lambda i: (i, 0)),
            pl.BlockSpec((BE, dp), lambda i: (i, 0)),
        ],
        out_shape=[
            jax.ShapeDtypeStruct((T_E, 2 * HID), jnp.float32),
            jax.ShapeDtypeStruct((T_E, dp), jnp.float32),
        ],
    )(*args)


def _mlp2_body(x_ref, w1_ref, b1_ref, w2_ref, b2_ref, out_ref):
    h = jnp.maximum(x_ref[...] @ w1_ref[...] + b1_ref[...], 0.0)
    out_ref[...] = jnp.maximum(h @ w2_ref[...] + b2_ref[...], 0.0)


def _mlp2(x, w1, b1, w2, b2):
    n, din = x.shape
    dh = w1.shape[1]
    dout = w2.shape[1]
    return pl.pallas_call(
        _mlp2_body,
        grid=(n // BN,),
        in_specs=[
            pl.BlockSpec((BN, din), lambda i: (i, 0)),
            pl.BlockSpec((din, dh), lambda i: (0, 0)),
            pl.BlockSpec((1, dh), lambda i: (0, 0)),
            pl.BlockSpec((dh, dout), lambda i: (0, 0)),
            pl.BlockSpec((1, dout), lambda i: (0, 0)),
        ],
        out_specs=pl.BlockSpec((BN, dout), lambda i: (i, 0)),
        out_shape=jax.ShapeDtypeStruct((n, dout), jnp.float32),
    )(x, w1, b1, w2, b2)


def _head_body(x_ref, h1_ref, c1_ref, h2_ref, c2_ref, wm_ref, bm_ref,
               wv_ref, bv_ref, mu_ref, lv_ref):
    h = jnp.maximum(x_ref[...] @ h1_ref[...] + c1_ref[...], 0.0)
    be = jnp.maximum(h @ h2_ref[...] + c2_ref[...], 0.0)
    mu_ref[...] = be @ wm_ref[...] + bm_ref[...]
    lv_ref[...] = be @ wv_ref[...] + bv_ref[...]


def _head(x, mv_params, mean_p, var_p):
    (h1, c1), (h2, c2) = mv_params
    wm, bm = mean_p
    wv, bv = var_p
    c1 = c1.reshape(1, -1)
    c2 = c2.reshape(1, -1)
    bm = bm.reshape(1, -1)
    bv = bv.reshape(1, -1)
    n = x.shape[0]
    return pl.pallas_call(
        _head_body,
        grid=(n // BN,),
        in_specs=[
            pl.BlockSpec((BN, 2 * EMB), lambda i: (i, 0)),
            pl.BlockSpec((2 * EMB, HID), lambda i: (0, 0)),
            pl.BlockSpec((1, HID), lambda i: (0, 0)),
            pl.BlockSpec((HID, 2 * EMB), lambda i: (0, 0)),
            pl.BlockSpec((1, 2 * EMB), lambda i: (0, 0)),
            pl.BlockSpec((2 * EMB, EMB), lambda i: (0, 0)),
            pl.BlockSpec((1, EMB), lambda i: (0, 0)),
            pl.BlockSpec((2 * EMB, EMB), lambda i: (0, 0)),
            pl.BlockSpec((1, EMB), lambda i: (0, 0)),
        ],
        out_specs=[
            pl.BlockSpec((BN, EMB), lambda i: (i, 0)),
            pl.BlockSpec((BN, EMB), lambda i: (i, 0)),
        ],
        out_shape=[
            jax.ShapeDtypeStruct((n, EMB), jnp.float32),
            jax.ShapeDtypeStruct((n, EMB), jnp.float32),
        ],
    )(x, h1, c1, h2, c2, wm, bm, wv, bv)


def _reorder_w2(w2, b2, dp):
    w2r = jnp.concatenate([w2[:, :HID], w2[:, HID + dp:], w2[:, HID:HID + dp]],
                          axis=1)
    b2r = jnp.concatenate([b2[:HID], b2[HID + dp:], b2[HID:HID + dp]])
    return w2r, b2r.reshape(1, -1)


def _layer_core(ab, preds, b1, w2, b2, net2, dp, aux):
    ia, ib, node_il, src_both, zeros128, inv = aux
    tab = ab.reshape(2 * O_N, HID)
    g1, g2 = _sc_gather2(tab, ia, ib)
    w2r, b2r = _reorder_w2(w2, b2, dp)
    scat, new_p = _edge_mlp(g1, g2, preds, b1.reshape(1, -1), w2r, b2r)
    pooled2 = _sc_scatter(scat.reshape(4 * T_E, 128), node_il, src_both,
                          zeros128)
    pooled = jnp.concatenate([pooled2[0], pooled2[1]], axis=1)[:O_N]
    pooled = pooled * inv
    (v1, c1), (v2, c2) = net2
    new_obj = _mlp2(pooled, v1, c1.reshape(1, -1), v2, c2.reshape(1, -1))
    return new_obj, new_p


def kernel(objs, triples, boxes_gt, shapes_gt, attributes, params):
    s = triples[:, 0].astype(jnp.int32)
    p = triples[:, 1].astype(jnp.int32)
    o = triples[:, 2].astype(jnp.int32)

    ia = 2 * s
    ib = 2 * o + 1
    node_il = jnp.stack([s, o], axis=1).reshape(-1)
    base4 = 4 * jnp.arange(T_E, dtype=jnp.int32)
    il0 = jnp.stack([base4, base4 + 2], axis=1).reshape(-1)
    src_both = jnp.stack([il0, il0 + 1], axis=0)
    zeros128 = jnp.zeros((NROWS_T, 128), jnp.float32)
    ones128 = jnp.ones((C, 128), jnp.float32)

    cnt2 = _sc_counts(node_il, ones128, zeros128)
    counts = cnt2[0, :O_N, 0] + cnt2[1, :O_N, 0]
    inv = (1.0 / jnp.maximum(counts, 1.0))[:, None]

    onehot_obj = (objs[:, None] == jnp.arange(64, dtype=objs.dtype)
                  ).astype(jnp.float32)
    onehot_p = (p[:, None] == jnp.arange(16, dtype=jnp.int32)
                ).astype(jnp.float32)

    emb_ob = jnp.concatenate(
        [params['emb_obj_box'], jnp.zeros((64 - 37, EMB), jnp.float32)])
    emb_os = jnp.concatenate(
        [params['emb_obj_shape'], jnp.zeros((64 - 37, EMB), jnp.float32)])
    wbe, bbe = params['box_embeddings']
    wse, bse = params['shape_embeddings']
    boxes_p = jnp.pad(boxes_gt, ((0, 0), (0, 2)))
    wbe_p = jnp.pad(wbe, ((0, 2), (0, 0)))

    aux = (ia, ib, node_il, src_both, zeros128, inv)

    def run_stack(layers, emb_tab, x2, w2feat, b2feat, pred0_tab):
        ob_vecs = None
        pb = None
        for li, layer in enumerate(layers):
            net1, net2 = layer
            (w1, b1), (w2, b2) = net1
            din = DIN
            w1s, w1p, w1o = w1[:din], w1[din:2 * din], w1[2 * din:]
            w1so = jnp.concatenate([w1s, w1o], axis=1)
            if li == 0:
                t1 = emb_tab @ w1so[:EMB]
                t2 = w2feat @ w1so[EMB:]
                crow = (b2feat @ w1so[EMB:]).reshape(1, -1)
                ab = _ab2(onehot_obj, t1, x2, t2, crow)
                preds = [(onehot_p, pred0_tab @ w1p)]
            else:
                ab = _dense1(ob_vecs, w1so)
                preds = [(pb, w1p)]
            ob_vecs, pb = _layer_core(ab, preds, b1, w2, b2, net2, DIN, aux)
        return ob_vecs, pb

    ob, pb = run_stack(params['gconv_ec_box'], emb_ob, boxes_p, wbe_p, bbe,
                       params['emb_pred_box'])
    osh, ps = run_stack(params['gconv_ec_shape'], emb_os, shapes_gt, wse, bse,
                        params['emb_pred_shape'])

    for layer in params['gconv_shared']:
        net1, net2 = layer
        (w1, b1), (w2, b2) = net1
        din = 2 * DIN
        w1s, w1p, w1o = w1[:din], w1[din:2 * din], w1[2 * din:]
        w1so = jnp.concatenate([w1s, w1o], axis=1)
        ab = _ab2(ob, w1so[:DIN], osh, w1so[DIN:],
                  jnp.zeros((1, 2 * HID), jnp.float32))
        preds = [(pb, w1p[:DIN]), (ps, w1p[DIN:])]
        obj_sh, pred_sh = _layer_core(ab, preds, b1, w2, b2, net2, 2 * DIN,
                                      aux)
        ob, osh = obj_sh[:, :DIN], obj_sh[:, DIN:]
        pb, ps = pred_sh[:, :DIN], pred_sh[:, DIN:]

    mu_box, logvar_box = _head(ob, params['box_mean_var'],
                               params['box_mean'][0], params['box_var'][0])
    mu_shape, logvar_shape = _head(osh, params['shape_mean_var'],
                                   params['shape_mean'][0], params['shape_var'][0])
    return mu_box, logvar_box, mu_shape, logvar_shape
